# Initial kernel scaffold; baseline (speedup 1.0000x reference)
import jax
import jax.numpy as jnp
from jax.experimental import pallas as pl


def kernel(logits, k, p, noise_u, no_top_k, no_top_p):
    raise NotImplementedError("write your pallas kernel here")


# speedup vs baseline: 29.1989x; 29.1989x over previous
import jax
import jax.numpy as jnp
from jax import lax
from jax.experimental import pallas as pl
from jax.experimental.pallas import tpu as pltpu

_R = 8

def _row_kernel(k_ref, p_ref, flags_ref, x_ref, noise_ref, out_ref):
    sign_bit = jnp.int32(-2147483648)
    low31 = jnp.int32(2147483647)
    i = pl.program_id(0)
    x = x_ref[0]
    n_rows, n_cols = x.shape
    kk = k_ref[i]
    pp = p_ref[i]
    skip_k = flags_ref[0] != 0
    skip_p = flags_ref[1] != 0

    bits = lax.bitcast_convert_type(x, jnp.int32)
    key = jnp.where(bits < 0, bits ^ low31, bits)

    def build(pred):
        cnt0 = pred(jnp.int32(0))
        t0 = jnp.where(cnt0, jnp.int32(0), sign_bit)

        def step(it, t):
            bit = jnp.left_shift(jnp.int32(1), jnp.int32(30) - it)
            cand = t | bit
            return jnp.where(pred(cand), cand, t)

        return lax.fori_loop(0, 31, step, t0)

    t1 = build(lambda cand: jnp.sum((key >= cand).astype(jnp.int32)) >= kk)
    surv_k = (key >= t1) | skip_k

    m = jnp.max(x)
    e = jnp.where(surv_k, jnp.exp(x - m), 0.0)
    z = jnp.sum(e)
    pz = pp * z

    t2 = build(lambda cand: jnp.sum(jnp.where(key > cand, e, 0.0)) >= pz)
    kmax = jnp.max(key)
    surv = surv_k & ((key > t2) | (key == kmax) | skip_p)

    q = -jnp.log1p(-noise_ref[0]) + 1e-10
    val = jnp.where(surv, e, 0.0) / q
    mx = jnp.max(val)
    r_iota = lax.broadcasted_iota(jnp.int32, (n_rows, n_cols), 0)
    c_iota = lax.broadcasted_iota(jnp.int32, (n_rows, n_cols), 1)
    flat = r_iota * n_cols + c_iota
    idx = jnp.min(jnp.where(val == mx, flat, n_rows * n_cols))
    out_ref[...] = jnp.full((1, 1, 1), idx, jnp.int32)


def kernel(logits, k, p, noise_u, no_top_k, no_top_p):
    b, v = logits.shape
    c = v // _R
    x3 = logits.reshape(b, _R, c)
    n3 = noise_u.reshape(b, _R, c)
    flags = jnp.stack([jnp.asarray(no_top_k, jnp.int32),
                       jnp.asarray(no_top_p, jnp.int32)])
    grid_spec = pltpu.PrefetchScalarGridSpec(
        num_scalar_prefetch=3,
        grid=(b,),
        in_specs=[
            pl.BlockSpec((1, _R, c), lambda i, *_: (i, 0, 0)),
            pl.BlockSpec((1, _R, c), lambda i, *_: (i, 0, 0)),
        ],
        out_specs=pl.BlockSpec((1, 1, 1), lambda i, *_: (i, 0, 0)),
    )
    out = pl.pallas_call(
        _row_kernel,
        grid_spec=grid_spec,
        out_shape=jax.ShapeDtypeStruct((b, 1, 1), jnp.int32),
        compiler_params=pltpu.CompilerParams(
            dimension_semantics=("parallel",)),
    )(k.astype(jnp.int32), p.astype(jnp.float32), flags, x3, n3)
    return out.reshape(-1)
